# baseline (speedup 1.0000x reference)
import jax
import jax.numpy as jnp
from jax.experimental import pallas as pl
from jax.experimental.pallas import tpu as pltpu

_N = 10000
_F = 128
_DEPTH = 3
_A = 1.0
_B = 1.0
_LB = -1.0
_RB = 1.0

_BM1 = 400
_BM = 1000
_NB = _N // _BM


def _pass1_kernel(coef_ref, adj_ref, x0b_ref, x0_ref,
                  x1_ref, x1b_ref, adjb_ref, cs_ref):
    adjb = adj_ref[...].astype(jnp.bfloat16)
    adjb_ref[...] = adjb
    y = jnp.dot(adjb, x0b_ref[...], preferred_element_type=jnp.float32)
    x1 = coef_ref[0] * y + coef_ref[1] * x0_ref[...]
    x1_ref[...] = x1
    x1b_ref[...] = x1.astype(jnp.bfloat16)
    part = jnp.concatenate(
        [jnp.sum(x0_ref[...], axis=0, keepdims=True),
         jnp.sum(x1, axis=0, keepdims=True)], axis=0)

    @pl.when(pl.program_id(0) == 0)
    def _init():
        cs_ref[...] = part

    @pl.when(pl.program_id(0) != 0)
    def _acc():
        cs_ref[...] = cs_ref[...] + part


def _pass1(coefs, adj, x0b, x0, bm):
    nb = _N // bm
    return pl.pallas_call(
        _pass1_kernel,
        grid=(nb,),
        in_specs=[
            pl.BlockSpec(memory_space=pltpu.SMEM),
            pl.BlockSpec((bm, _N), lambda i: (i, 0)),
            pl.BlockSpec((_N, _F), lambda i: (0, 0)),
            pl.BlockSpec((bm, _F), lambda i: (i, 0)),
        ],
        out_specs=[
            pl.BlockSpec((bm, _F), lambda i: (i, 0)),
            pl.BlockSpec((bm, _F), lambda i: (i, 0)),
            pl.BlockSpec((bm, _N), lambda i: (i, 0)),
            pl.BlockSpec((2, _F), lambda i: (0, 0)),
        ],
        out_shape=[
            jax.ShapeDtypeStruct((_N, _F), jnp.float32),
            jax.ShapeDtypeStruct((_N, _F), jnp.bfloat16),
            jax.ShapeDtypeStruct((_N, _N), jnp.bfloat16),
            jax.ShapeDtypeStruct((2, _F), jnp.float32),
        ],
    )(coefs, adj, x0b, x0)


def _b16(v):
    return v.astype(jnp.bfloat16).astype(jnp.float32)


def _mega_kernel(coef_ref, adjb_ref, x1b_ref, x1bw_ref, x1_ref, x0_ref,
                 x0b_ref, cs01_ref, lwb_ref, lb_ref, fwb_ref, fb_ref,
                 out_ref, x23f_s, x2b_s, x2b3_s, cs_s):
    s = pl.program_id(0)
    i = pl.program_id(1)
    b = jnp.where(s == 1, _NB - 1 - i, i)
    rows = pl.ds(b * _BM, _BM)

    @pl.when(s == 0)
    def _pass2():
        y = jnp.dot(adjb_ref[...], x1b_ref[...],
                    preferred_element_type=jnp.float32)
        x2 = (coef_ref[0] * y + coef_ref[1] * x1_ref[...]
              + coef_ref[2] * x0_ref[...])
        x23f_s[rows, :] = x2
        x2b3_s[i] = x2.astype(jnp.bfloat16)
        part = jnp.sum(x2, axis=0, keepdims=True)
        prev = jnp.where(i == 0, jnp.zeros_like(part), cs_s[0:1, :])
        cs_s[0:1, :] = prev + part

    @pl.when((s == 1) & (i == 0))
    def _cast_x2():
        x2b_s[...] = x23f_s[...].astype(jnp.bfloat16)

    @pl.when(s == 1)
    def _pass3():
        y = jnp.dot(adjb_ref[...], x2b_s[...],
                    preferred_element_type=jnp.float32)
        x3 = (coef_ref[3] * y + coef_ref[4] * x23f_s[rows, :]
              + coef_ref[5] * x1_ref[...])
        x23f_s[rows, :] = x3
        part = jnp.sum(x3, axis=0, keepdims=True)
        prev = jnp.where(i == 0, jnp.zeros_like(part), cs_s[1:2, :])
        cs_s[1:2, :] = prev + part

    @pl.when(s == 2)
    def _fuse():
        q = jnp.concatenate([cs01_ref[...], cs_s[0:2, :]],
                            axis=0) * (1.0 / _N)
        xks = (x0b_ref[...], x1bw_ref[...],
               x2b3_s[i],
               x23f_s[rows, :].astype(jnp.bfloat16))
        qb = q.astype(jnp.bfloat16)
        krow = jax.lax.broadcasted_iota(jnp.int32, (_DEPTH + 1, _F), 0)
        xprojs = []
        logits = None
        for k, xk in enumerate(xks):
            xp = jax.lax.dot_general(
                xk, lwb_ref[k],
                (((1,), (1,)), ((), ())),
                preferred_element_type=jnp.float32,
            ) + lb_ref[k][None, :]
            xpb = xp.astype(jnp.bfloat16)
            xprojs.append(xpb)
            qmask = jnp.where(krow == k, qb, jnp.zeros_like(qb))
            part = jax.lax.dot_general(
                xpb, qmask,
                (((1,), (1,)), ((), ())),
                preferred_element_type=jnp.float32,
            )
            logits = part if logits is None else logits + part
        t = jnp.tanh(logits)
        m = jnp.max(t, axis=1, keepdims=True)
        e = jnp.exp(t - m)
        w = e / jnp.sum(e, axis=1, keepdims=True)
        wb = w.astype(jnp.bfloat16).astype(jnp.float32)
        acc = None
        for k in range(4):
            term = wb[:, k:k + 1] * xprojs[k].astype(jnp.float32)
            acc = term if acc is None else acc + term
        out_ref[...] = jax.lax.dot_general(
            acc.astype(jnp.bfloat16), fwb_ref[...],
            (((1,), (1,)), ((), ())),
            preferred_element_type=jnp.float32,
        ) + fb_ref[...]


def _mega(coefs, adjb, x1b, x1, x0, x0b, cs01, lin_Wb, lin_b, fc_Wb, fc_b):
    last = _NB - 1
    return pl.pallas_call(
        _mega_kernel,
        grid=(3, _NB),
        in_specs=[
            pl.BlockSpec(memory_space=pltpu.SMEM),
            pl.BlockSpec((_BM, _N), lambda s, i: (
                jnp.where(s == 0, i, jnp.where(s == 1, last - i, 0)), 0)),
            pl.BlockSpec((_N, _F), lambda s, i: (0, 0)),
            pl.BlockSpec((_BM, _F), lambda s, i: (jnp.where(s == 2, i, 0), 0)),
            pl.BlockSpec((_BM, _F), lambda s, i: (
                jnp.where(s == 0, i, jnp.where(s == 1, last - i, 0)), 0)),
            pl.BlockSpec((_BM, _F), lambda s, i: (jnp.where(s == 0, i, last), 0)),
            pl.BlockSpec((_BM, _F), lambda s, i: (jnp.where(s == 2, i, 0), 0)),
            pl.BlockSpec((2, _F), lambda s, i: (0, 0)),
            pl.BlockSpec((_DEPTH + 1, _F, _F), lambda s, i: (0, 0, 0)),
            pl.BlockSpec((_DEPTH + 1, _F), lambda s, i: (0, 0)),
            pl.BlockSpec((_F, _F), lambda s, i: (0, 0)),
            pl.BlockSpec((1, _F), lambda s, i: (0, 0)),
        ],
        out_specs=pl.BlockSpec(
            (_BM, _F), lambda s, i: (jnp.where(s == 2, i, _NB), 0)),
        out_shape=jax.ShapeDtypeStruct((_N + _BM, _F), jnp.float32),
        scratch_shapes=[
            pltpu.VMEM((_N, _F), jnp.float32),
            pltpu.VMEM((_N, _F), jnp.bfloat16),
            pltpu.VMEM((_NB, _BM, _F), jnp.bfloat16),
            pltpu.VMEM((8, _F), jnp.float32),
        ],
    )(coefs, adjb, x1b, x1b, x1, x0, x0b, cs01, lin_Wb, lin_b, fc_Wb,
      fc_b.reshape(1, _F))


@jax.jit
def kernel(x, adj, alphas_raw, lin_W, lin_b, fc_W, fc_b):
    alphas = jnp.tanh(alphas_raw)
    a, b, l, r = _A, _B, _LB, _RB

    c1 = ((a - b) / 2 - (a + b + 2) / 2 * (l + r) / (r - l)) * alphas[0]
    c2 = ((a + b + 2) / (r - l)) * alphas[0]

    def rec_coefs(L):
        coef_l = 2 * L * (L + a + b) * (2 * L - 2 + a + b)
        coef_lm1_1 = (2 * L + a + b - 1) * (2 * L + a + b) * (2 * L + a + b - 2)
        coef_lm1_2 = (2 * L + a + b - 1) * (a ** 2 - b ** 2)
        coef_lm2 = 2 * (L - 1 + a) * (L - 1 + b) * (2 * L + a + b)
        tmp1 = alphas[L - 1] * (coef_lm1_1 / coef_l)
        tmp2 = alphas[L - 1] * (coef_lm1_2 / coef_l)
        tmp3 = alphas[L - 1] * alphas[L - 2] * (coef_lm2 / coef_l)
        tmp1_2 = tmp1 * (2 / (r - l))
        tmp2_2 = tmp1 * ((r + l) / (r - l)) + tmp2
        return tmp1_2, -tmp2_2, -tmp3

    x0 = x
    x0b = x0.astype(jnp.bfloat16)
    x1, x1b, adjb, cs01 = _pass1(jnp.stack([c2, c1]), adj, x0b, x0, _BM1)
    ca2, cb2, cc2 = rec_coefs(2)
    ca3, cb3, cc3 = rec_coefs(3)
    coefs = jnp.stack([ca2, cb2, cc2, ca3, cb3, cc3])
    out = _mega(coefs, adjb, x1b, x1, x0, x0b, cs01,
                lin_W.astype(jnp.bfloat16), lin_b,
                fc_W.astype(jnp.bfloat16), fc_b)
    return out[:_N]


# speedup vs baseline: 1.0523x; 1.0523x over previous
import jax
import jax.numpy as jnp
from jax.experimental import pallas as pl
from jax.experimental.pallas import tpu as pltpu

_N = 10000
_F = 128
_DEPTH = 3
_A = 1.0
_B = 1.0
_LB = -1.0
_RB = 1.0

_CSPLIT = 4992
_BM1 = 400
_BM = 1000
_NB = _N // _BM


def _pass1_kernel(coef_ref, adj_ref, x0b_ref, x0_ref,
                  x1_ref, x1b_ref, adjbA_ref, adjbB_ref, cs_ref):
    adjb = adj_ref[...].astype(jnp.bfloat16)
    adjbA_ref[...] = adjb[:, :_CSPLIT]
    adjbB_ref[...] = adjb[:, _CSPLIT:]
    y = jnp.dot(adjb, x0b_ref[...], preferred_element_type=jnp.float32)
    x1 = coef_ref[0] * y + coef_ref[1] * x0_ref[...]
    x1_ref[...] = x1
    x1b_ref[...] = x1.astype(jnp.bfloat16)
    part = jnp.concatenate(
        [jnp.sum(x0_ref[...], axis=0, keepdims=True),
         jnp.sum(x1, axis=0, keepdims=True)], axis=0)

    @pl.when(pl.program_id(0) == 0)
    def _init():
        cs_ref[...] = part

    @pl.when(pl.program_id(0) != 0)
    def _acc():
        cs_ref[...] = cs_ref[...] + part


def _pass1(coefs, adj, x0b, x0, bm):
    nb = _N // bm
    return pl.pallas_call(
        _pass1_kernel,
        grid=(nb,),
        in_specs=[
            pl.BlockSpec(memory_space=pltpu.SMEM),
            pl.BlockSpec((bm, _N), lambda i: (i, 0)),
            pl.BlockSpec((_N, _F), lambda i: (0, 0)),
            pl.BlockSpec((bm, _F), lambda i: (i, 0)),
        ],
        out_specs=[
            pl.BlockSpec((bm, _F), lambda i: (i, 0)),
            pl.BlockSpec((bm, _F), lambda i: (i, 0)),
            pl.BlockSpec((bm, _CSPLIT), lambda i: (i, 0)),
            pl.BlockSpec((bm, _N - _CSPLIT), lambda i: (i, 0)),
            pl.BlockSpec((2, _F), lambda i: (0, 0)),
        ],
        out_shape=[
            jax.ShapeDtypeStruct((_N, _F), jnp.float32),
            jax.ShapeDtypeStruct((_N, _F), jnp.bfloat16),
            jax.ShapeDtypeStruct((_N, _CSPLIT), jnp.bfloat16),
            jax.ShapeDtypeStruct((_N, _N - _CSPLIT), jnp.bfloat16),
            jax.ShapeDtypeStruct((2, _F), jnp.float32),
        ],
    )(coefs, adj, x0b, x0)


def _b16(v):
    return v.astype(jnp.bfloat16).astype(jnp.float32)


def _mega_kernel(coef_ref, adjbA_ref, adjbB_ref, x1b_ref, x1bw_ref,
                 x1_ref, x0_ref, x0b_ref, cs01_ref, lwb_ref, lb_ref,
                 fwb_ref, fb_ref, out_ref, x23f_s, x2b_s, x2b3_s, cs_s):
    s = pl.program_id(0)
    i = pl.program_id(1)
    b = jnp.where(s == 1, _NB - 1 - i, i)
    rows = pl.ds(b * _BM, _BM)

    @pl.when(s == 0)
    def _pass2():
        xb = x1b_ref[...]
        y = (jnp.dot(adjbA_ref[...], xb[:_CSPLIT],
                     preferred_element_type=jnp.float32)
             + jnp.dot(adjbB_ref[...], xb[_CSPLIT:],
                       preferred_element_type=jnp.float32))
        x2 = (coef_ref[0] * y + coef_ref[1] * x1_ref[...]
              + coef_ref[2] * x0_ref[...])
        x23f_s[rows, :] = x2
        x2b3_s[i] = x2.astype(jnp.bfloat16)
        part = jnp.sum(x2, axis=0, keepdims=True)
        prev = jnp.where(i == 0, jnp.zeros_like(part), cs_s[0:1, :])
        cs_s[0:1, :] = prev + part

    @pl.when((s == 1) & (i == 0))
    def _cast_x2():
        x2b_s[...] = x23f_s[...].astype(jnp.bfloat16)

    @pl.when(s == 1)
    def _pass3():
        y = (jnp.dot(adjbA_ref[...], x2b_s[:_CSPLIT],
                      preferred_element_type=jnp.float32)
             + jnp.dot(adjbB_ref[...], x2b_s[_CSPLIT:],
                       preferred_element_type=jnp.float32))
        x3 = (coef_ref[3] * y + coef_ref[4] * x23f_s[rows, :]
              + coef_ref[5] * x1_ref[...])
        x23f_s[rows, :] = x3
        part = jnp.sum(x3, axis=0, keepdims=True)
        prev = jnp.where(i == 0, jnp.zeros_like(part), cs_s[1:2, :])
        cs_s[1:2, :] = prev + part

    @pl.when(s == 2)
    def _fuse():
        q = jnp.concatenate([cs01_ref[...], cs_s[0:2, :]],
                            axis=0) * (1.0 / _N)
        xks = (x0b_ref[...], x1bw_ref[...],
               x2b3_s[i],
               x23f_s[rows, :].astype(jnp.bfloat16))
        xprojs = []
        logits = []
        for k, xk in enumerate(xks):
            xp = jax.lax.dot_general(
                xk, lwb_ref[k],
                (((1,), (1,)), ((), ())),
                preferred_element_type=jnp.float32,
            ) + lb_ref[k][None, :]
            xpb = _b16(xp)
            qb = _b16(q[k])
            t = jnp.tanh(jnp.sum(qb[None, :] * xpb, axis=1, keepdims=True))
            xprojs.append(xpb)
            logits.append(t)
        m = jnp.maximum(jnp.maximum(logits[0], logits[1]),
                        jnp.maximum(logits[2], logits[3]))
        es = [jnp.exp(t - m) for t in logits]
        denom = es[0] + es[1] + es[2] + es[3]
        acc = None
        for k in range(4):
            wb = _b16(es[k] / denom)
            term = wb * xprojs[k]
            acc = term if acc is None else acc + term
        out_ref[...] = jax.lax.dot_general(
            acc.astype(jnp.bfloat16), fwb_ref[...],
            (((1,), (1,)), ((), ())),
            preferred_element_type=jnp.float32,
        ) + fb_ref[...]


def _mega(coefs, adjbA, adjbB, x1b, x1, x0, x0b, cs01, lin_Wb, lin_b,
          fc_Wb, fc_b):
    last = _NB - 1
    return pl.pallas_call(
        _mega_kernel,
        grid=(3, _NB),
        in_specs=[
            pl.BlockSpec(memory_space=pltpu.SMEM),
            pl.BlockSpec((_BM, _CSPLIT), lambda s, i: (
                jnp.where(s == 0, i, jnp.where(s == 1, last - i, 0)), 0)),
            pl.BlockSpec((_BM, _N - _CSPLIT), lambda s, i: (
                jnp.where(s == 0, i, jnp.where(s == 1, last - i, 0)), 0)),
            pl.BlockSpec((_N, _F), lambda s, i: (0, 0)),
            pl.BlockSpec((_BM, _F), lambda s, i: (jnp.where(s == 2, i, 0), 0)),
            pl.BlockSpec((_BM, _F), lambda s, i: (
                jnp.where(s == 0, i, jnp.where(s == 1, last - i, 0)), 0)),
            pl.BlockSpec((_BM, _F), lambda s, i: (jnp.where(s == 0, i, last), 0)),
            pl.BlockSpec((_BM, _F), lambda s, i: (jnp.where(s == 2, i, 0), 0)),
            pl.BlockSpec((2, _F), lambda s, i: (0, 0)),
            pl.BlockSpec((_DEPTH + 1, _F, _F), lambda s, i: (0, 0, 0)),
            pl.BlockSpec((_DEPTH + 1, _F), lambda s, i: (0, 0)),
            pl.BlockSpec((_F, _F), lambda s, i: (0, 0)),
            pl.BlockSpec((1, _F), lambda s, i: (0, 0)),
        ],
        out_specs=pl.BlockSpec(
            (_BM, _F), lambda s, i: (jnp.where(s == 2, i, _NB), 0)),
        out_shape=jax.ShapeDtypeStruct((_N + _BM, _F), jnp.float32),
        scratch_shapes=[
            pltpu.VMEM((_N, _F), jnp.float32),
            pltpu.VMEM((_N, _F), jnp.bfloat16),
            pltpu.VMEM((_NB, _BM, _F), jnp.bfloat16),
            pltpu.VMEM((8, _F), jnp.float32),
        ],
    )(coefs, adjbA, adjbB, x1b, x1b, x1, x0, x0b, cs01, lin_Wb, lin_b,
      fc_Wb, fc_b.reshape(1, _F))


@jax.jit
def kernel(x, adj, alphas_raw, lin_W, lin_b, fc_W, fc_b):
    alphas = jnp.tanh(alphas_raw)
    a, b, l, r = _A, _B, _LB, _RB

    c1 = ((a - b) / 2 - (a + b + 2) / 2 * (l + r) / (r - l)) * alphas[0]
    c2 = ((a + b + 2) / (r - l)) * alphas[0]

    def rec_coefs(L):
        coef_l = 2 * L * (L + a + b) * (2 * L - 2 + a + b)
        coef_lm1_1 = (2 * L + a + b - 1) * (2 * L + a + b) * (2 * L + a + b - 2)
        coef_lm1_2 = (2 * L + a + b - 1) * (a ** 2 - b ** 2)
        coef_lm2 = 2 * (L - 1 + a) * (L - 1 + b) * (2 * L + a + b)
        tmp1 = alphas[L - 1] * (coef_lm1_1 / coef_l)
        tmp2 = alphas[L - 1] * (coef_lm1_2 / coef_l)
        tmp3 = alphas[L - 1] * alphas[L - 2] * (coef_lm2 / coef_l)
        tmp1_2 = tmp1 * (2 / (r - l))
        tmp2_2 = tmp1 * ((r + l) / (r - l)) + tmp2
        return tmp1_2, -tmp2_2, -tmp3

    x0 = x
    x0b = x0.astype(jnp.bfloat16)
    x1, x1b, adjbA, adjbB, cs01 = _pass1(jnp.stack([c2, c1]), adj, x0b,
                                         x0, _BM1)
    ca2, cb2, cc2 = rec_coefs(2)
    ca3, cb3, cc3 = rec_coefs(3)
    coefs = jnp.stack([ca2, cb2, cc2, ca3, cb3, cc3])
    out = _mega(coefs, adjbA, adjbB, x1b, x1, x0, x0b, cs01,
                lin_W.astype(jnp.bfloat16), lin_b,
                fc_W.astype(jnp.bfloat16), fc_b)
    return out[:_N]


# speedup vs baseline: 1.0716x; 1.0184x over previous
import jax
import jax.numpy as jnp
from jax.experimental import pallas as pl
from jax.experimental.pallas import tpu as pltpu

_N = 10000
_F = 128
_DEPTH = 3
_A = 1.0
_B = 1.0
_LB = -1.0
_RB = 1.0

_CSPLIT = 4992
_BM1 = 400
_BM = 1000
_NB = _N // _BM


def _pass1_kernel(coef_ref, adj_ref, x0b_ref, x0_ref,
                  x1_ref, x1b_ref, adjbA_ref, adjbB_ref, cs_ref):
    adjb = adj_ref[...].astype(jnp.bfloat16)
    adjbA_ref[...] = adjb[:, :_CSPLIT]
    adjbB_ref[...] = adjb[:, _CSPLIT:]
    y = jnp.dot(adjb, x0b_ref[...], preferred_element_type=jnp.float32)
    x1 = coef_ref[0] * y + coef_ref[1] * x0_ref[...]
    x1_ref[...] = x1
    x1b_ref[...] = x1.astype(jnp.bfloat16)
    part = jnp.concatenate(
        [jnp.sum(x0_ref[...], axis=0, keepdims=True),
         jnp.sum(x1, axis=0, keepdims=True)], axis=0)

    @pl.when(pl.program_id(0) == 0)
    def _init():
        cs_ref[...] = part

    @pl.when(pl.program_id(0) != 0)
    def _acc():
        cs_ref[...] = cs_ref[...] + part


def _pass1(coefs, adj, x0b, x0, bm):
    nb = _N // bm
    return pl.pallas_call(
        _pass1_kernel,
        grid=(nb,),
        in_specs=[
            pl.BlockSpec(memory_space=pltpu.SMEM),
            pl.BlockSpec((bm, _N), lambda i: (i, 0)),
            pl.BlockSpec((_N, _F), lambda i: (0, 0)),
            pl.BlockSpec((bm, _F), lambda i: (i, 0)),
        ],
        out_specs=[
            pl.BlockSpec((bm, _F), lambda i: (i, 0)),
            pl.BlockSpec((bm, _F), lambda i: (i, 0)),
            pl.BlockSpec((bm, _CSPLIT), lambda i: (i, 0)),
            pl.BlockSpec((bm, _N - _CSPLIT), lambda i: (i, 0)),
            pl.BlockSpec((2, _F), lambda i: (0, 0)),
        ],
        out_shape=[
            jax.ShapeDtypeStruct((_N, _F), jnp.float32),
            jax.ShapeDtypeStruct((_N, _F), jnp.bfloat16),
            jax.ShapeDtypeStruct((_N, _CSPLIT), jnp.bfloat16),
            jax.ShapeDtypeStruct((_N, _N - _CSPLIT), jnp.bfloat16),
            jax.ShapeDtypeStruct((2, _F), jnp.float32),
        ],
    )(coefs, adj, x0b, x0)


def _b16(v):
    return v.astype(jnp.bfloat16).astype(jnp.float32)


def _mega_kernel(coef_ref, adjbA_ref, adjbB_ref, x1b_ref, x1bw_ref,
                 x1_ref, x0_ref, x0b_ref, cs01_ref, lwb_ref, lb_ref,
                 fwb_ref, fb_ref, out_ref, x23f_s, xp2_s, xp3_s, cs_s):
    s = pl.program_id(0)
    i = pl.program_id(1)
    b = jnp.where(s == 1, _NB - 1 - i, i)
    rows = pl.ds(b * _BM, _BM)

    @pl.when(s == 0)
    def _pass2():
        xb = x1b_ref[...]
        y = (jnp.dot(adjbA_ref[...], xb[:_CSPLIT],
                     preferred_element_type=jnp.float32)
             + jnp.dot(adjbB_ref[...], xb[_CSPLIT:],
                       preferred_element_type=jnp.float32))
        x2 = (coef_ref[0] * y + coef_ref[1] * x1_ref[...]
              + coef_ref[2] * x0_ref[...])
        x23f_s[rows, :] = x2
        part = jnp.sum(x2, axis=0, keepdims=True)
        prev = jnp.where(i == 0, jnp.zeros_like(part), cs_s[0:1, :])
        cs_s[0:1, :] = prev + part

    @pl.when(s == 1)
    def _pass3():
        x2ball = x23f_s[...].astype(jnp.bfloat16)
        y = (jnp.dot(adjbA_ref[...], x2ball[:_CSPLIT],
                      preferred_element_type=jnp.float32)
             + jnp.dot(adjbB_ref[...], x2ball[_CSPLIT:],
                       preferred_element_type=jnp.float32))
        x2 = x23f_s[rows, :]
        x3 = (coef_ref[3] * y + coef_ref[4] * x2
              + coef_ref[5] * x1_ref[...])
        part = jnp.sum(x3, axis=0, keepdims=True)
        prev = jnp.where(i == 0, jnp.zeros_like(part), cs_s[1:2, :])
        cs_s[1:2, :] = prev + part
        for k, xkb in ((2, x2.astype(jnp.bfloat16)),
                       (3, x3.astype(jnp.bfloat16))):
            xp = jax.lax.dot_general(
                xkb, lwb_ref[k],
                (((1,), (1,)), ((), ())),
                preferred_element_type=jnp.float32,
            ) + lb_ref[k][None, :]
            xps = xp2_s if k == 2 else xp3_s
            xps[b] = xp.astype(jnp.bfloat16)

    @pl.when(s == 2)
    def _fuse():
        q = jnp.concatenate([cs01_ref[...], cs_s[0:2, :]],
                            axis=0) * (1.0 / _N)
        xprojs = []
        logits = []
        for k in range(4):
            if k < 2:
                xk = x0b_ref[...] if k == 0 else x1bw_ref[...]
                xp = jax.lax.dot_general(
                    xk, lwb_ref[k],
                    (((1,), (1,)), ((), ())),
                    preferred_element_type=jnp.float32,
                ) + lb_ref[k][None, :]
                xpb = _b16(xp)
            else:
                xps = xp2_s if k == 2 else xp3_s
                xpb = xps[i].astype(jnp.float32)
            qb = _b16(q[k])
            t = jnp.tanh(jnp.sum(qb[None, :] * xpb, axis=1, keepdims=True))
            xprojs.append(xpb)
            logits.append(t)
        m = jnp.maximum(jnp.maximum(logits[0], logits[1]),
                        jnp.maximum(logits[2], logits[3]))
        es = [jnp.exp(t - m) for t in logits]
        denom = es[0] + es[1] + es[2] + es[3]
        acc = None
        for k in range(4):
            wb = _b16(es[k] / denom)
            term = wb * xprojs[k]
            acc = term if acc is None else acc + term
        out_ref[...] = jax.lax.dot_general(
            acc.astype(jnp.bfloat16), fwb_ref[...],
            (((1,), (1,)), ((), ())),
            preferred_element_type=jnp.float32,
        ) + fb_ref[...]


def _mega(coefs, adjbA, adjbB, x1b, x1, x0, x0b, cs01, lin_Wb, lin_b,
          fc_Wb, fc_b):
    last = _NB - 1
    return pl.pallas_call(
        _mega_kernel,
        grid=(3, _NB),
        in_specs=[
            pl.BlockSpec(memory_space=pltpu.SMEM),
            pl.BlockSpec((_BM, _CSPLIT), lambda s, i: (
                jnp.where(s == 0, i, jnp.where(s == 1, last - i, 0)), 0)),
            pl.BlockSpec((_BM, _N - _CSPLIT), lambda s, i: (
                jnp.where(s == 0, i, jnp.where(s == 1, last - i, 0)), 0)),
            pl.BlockSpec((_N, _F), lambda s, i: (0, 0)),
            pl.BlockSpec((_BM, _F), lambda s, i: (jnp.where(s == 2, i, 0), 0)),
            pl.BlockSpec((_BM, _F), lambda s, i: (
                jnp.where(s == 0, i, jnp.where(s == 1, last - i, 0)), 0)),
            pl.BlockSpec((_BM, _F), lambda s, i: (jnp.where(s == 0, i, last), 0)),
            pl.BlockSpec((_BM, _F), lambda s, i: (jnp.where(s == 2, i, 0), 0)),
            pl.BlockSpec((2, _F), lambda s, i: (0, 0)),
            pl.BlockSpec((_DEPTH + 1, _F, _F), lambda s, i: (0, 0, 0)),
            pl.BlockSpec((_DEPTH + 1, _F), lambda s, i: (0, 0)),
            pl.BlockSpec((_F, _F), lambda s, i: (0, 0)),
            pl.BlockSpec((1, _F), lambda s, i: (0, 0)),
        ],
        out_specs=pl.BlockSpec(
            (_BM, _F), lambda s, i: (jnp.where(s == 2, i, _NB), 0)),
        out_shape=jax.ShapeDtypeStruct((_N + _BM, _F), jnp.float32),
        scratch_shapes=[
            pltpu.VMEM((_N, _F), jnp.float32),
            pltpu.VMEM((_NB, _BM, _F), jnp.bfloat16),
            pltpu.VMEM((_NB, _BM, _F), jnp.bfloat16),
            pltpu.VMEM((8, _F), jnp.float32),
        ],
    )(coefs, adjbA, adjbB, x1b, x1b, x1, x0, x0b, cs01, lin_Wb, lin_b,
      fc_Wb, fc_b.reshape(1, _F))


@jax.jit
def kernel(x, adj, alphas_raw, lin_W, lin_b, fc_W, fc_b):
    alphas = jnp.tanh(alphas_raw)
    a, b, l, r = _A, _B, _LB, _RB

    c1 = ((a - b) / 2 - (a + b + 2) / 2 * (l + r) / (r - l)) * alphas[0]
    c2 = ((a + b + 2) / (r - l)) * alphas[0]

    def rec_coefs(L):
        coef_l = 2 * L * (L + a + b) * (2 * L - 2 + a + b)
        coef_lm1_1 = (2 * L + a + b - 1) * (2 * L + a + b) * (2 * L + a + b - 2)
        coef_lm1_2 = (2 * L + a + b - 1) * (a ** 2 - b ** 2)
        coef_lm2 = 2 * (L - 1 + a) * (L - 1 + b) * (2 * L + a + b)
        tmp1 = alphas[L - 1] * (coef_lm1_1 / coef_l)
        tmp2 = alphas[L - 1] * (coef_lm1_2 / coef_l)
        tmp3 = alphas[L - 1] * alphas[L - 2] * (coef_lm2 / coef_l)
        tmp1_2 = tmp1 * (2 / (r - l))
        tmp2_2 = tmp1 * ((r + l) / (r - l)) + tmp2
        return tmp1_2, -tmp2_2, -tmp3

    x0 = x
    x0b = x0.astype(jnp.bfloat16)
    x1, x1b, adjbA, adjbB, cs01 = _pass1(jnp.stack([c2, c1]), adj, x0b,
                                         x0, _BM1)
    ca2, cb2, cc2 = rec_coefs(2)
    ca3, cb3, cc3 = rec_coefs(3)
    coefs = jnp.stack([ca2, cb2, cc2, ca3, cb3, cc3])
    out = _mega(coefs, adjbA, adjbB, x1b, x1, x0, x0b, cs01,
                lin_W.astype(jnp.bfloat16), lin_b,
                fc_W.astype(jnp.bfloat16), fc_b)
    return out[:_N]
